# baseline (speedup 1.0000x reference)
import functools

import jax
import jax.numpy as jnp
from jax import lax
from jax.experimental import pallas as pl
from jax.experimental.pallas import tpu as pltpu
from jax.experimental.pallas import tpu_sc as plsc

_NC, _NS, _L = 2, 16, 16
_NW = _NC * _NS
_CS = 4


def kernel(x, pos_table):
    S, B, D = x.shape
    rows_per_w = S // _NW
    n_chunks = rows_per_w // _CS
    d_vecs = D // _L

    mesh = plsc.VectorSubcoreMesh(core_axis_name="c", subcore_axis_name="s")

    @functools.partial(
        pl.kernel,
        out_type=jax.ShapeDtypeStruct((S, B, D), x.dtype),
        mesh=mesh,
        scratch_types=[
            pltpu.VMEM((_CS, B, D), jnp.float32),
            pltpu.VMEM((_CS, B, D), jnp.float32),
            pltpu.VMEM((_CS, D), jnp.float32),
            pltpu.VMEM((_CS, D), jnp.float32),
            pltpu.VMEM((_CS, B, D), jnp.float32),
            pltpu.VMEM((_CS, B, D), jnp.float32),
            pltpu.SemaphoreType.DMA,
            pltpu.SemaphoreType.DMA,
            pltpu.SemaphoreType.DMA,
            pltpu.SemaphoreType.DMA,
        ],
    )
    def run(x_hbm, pos_hbm, out_hbm, xv0, xv1, pv0, pv1, ov0, ov1,
            si0, si1, so0, so1):
        wid = lax.axis_index("c") * _NS + lax.axis_index("s")
        base = wid * rows_per_w
        X, P, O = (xv0, xv1), (pv0, pv1), (ov0, ov1)
        SI, SO = (si0, si1), (so0, so1)

        def start_in(k, b):
            row0 = base + k * _CS
            pltpu.async_copy(x_hbm.at[pl.ds(row0, _CS)], X[b], SI[b])
            pltpu.async_copy(pos_hbm.at[pl.ds(row0, _CS)], P[b], SI[b])

        def wait_in(b):
            pltpu.make_async_copy(x_hbm.at[pl.ds(0, _CS)], X[b], SI[b]).wait()
            pltpu.make_async_copy(pos_hbm.at[pl.ds(0, _CS)], P[b], SI[b]).wait()

        def start_out(k, b):
            row0 = base + k * _CS
            pltpu.async_copy(O[b], out_hbm.at[pl.ds(row0, _CS)], SO[b])

        def wait_out(b):
            pltpu.make_async_copy(O[b], out_hbm.at[pl.ds(0, _CS)], SO[b]).wait()

        def compute(b):
            xv, pv, ov = X[b], P[b], O[b]

            @plsc.parallel_loop(0, _CS * d_vecs, unroll=8)
            def _(j):
                sl = j // d_vecs
                off = pl.multiple_of((j % d_vecs) * _L, _L)
                pvec = pv[sl, pl.ds(off, _L)]
                for bb in range(B):
                    ov[sl, bb, pl.ds(off, _L)] = (
                        xv[sl, bb, pl.ds(off, _L)] + pvec
                    )

        start_in(0, 0)
        start_in(1, 1)

        def pair(k2, carry):
            for b in range(2):
                k = 2 * k2 + b
                wait_in(b)

                @pl.when(k >= 2)
                def _():
                    wait_out(b)

                compute(b)
                start_out(k, b)

                @pl.when(k + 2 < n_chunks)
                def _():
                    start_in(k + 2, b)
            return carry

        lax.fori_loop(0, n_chunks // 2, pair, 0)
        wait_out(0)
        wait_out(1)

    return run(x, pos_table[:S])


# speedup vs baseline: 1.0036x; 1.0036x over previous
import functools

import jax
import jax.numpy as jnp
from jax import lax
from jax.experimental import pallas as pl
from jax.experimental.pallas import tpu as pltpu
from jax.experimental.pallas import tpu_sc as plsc

_NC, _NS, _L = 2, 16, 16
_NW = _NC * _NS
_CS = 4


def kernel(x, pos_table):
    S, B, D = x.shape
    rows_per_w = S // _NW
    n_chunks = rows_per_w // _CS
    d_vecs = D // _L

    mesh = plsc.VectorSubcoreMesh(core_axis_name="c", subcore_axis_name="s")

    @functools.partial(
        pl.kernel,
        out_type=jax.ShapeDtypeStruct((S, B, D), x.dtype),
        mesh=mesh,
        scratch_types=[
            pltpu.VMEM((_CS, B, D), jnp.float32),
            pltpu.VMEM((_CS, B, D), jnp.float32),
            pltpu.VMEM((_CS, D), jnp.float32),
            pltpu.VMEM((_CS, D), jnp.float32),
            pltpu.VMEM((_CS, B, D), jnp.float32),
            pltpu.VMEM((_CS, B, D), jnp.float32),
            pltpu.SemaphoreType.DMA,
            pltpu.SemaphoreType.DMA,
            pltpu.SemaphoreType.DMA,
            pltpu.SemaphoreType.DMA,
        ],
    )
    def run(x_hbm, pos_hbm, out_hbm, xv0, xv1, pv0, pv1, ov0, ov1,
            si0, si1, so0, so1):
        wid = lax.axis_index("s") * _NC + lax.axis_index("c")
        base = wid * rows_per_w
        X, P, O = (xv0, xv1), (pv0, pv1), (ov0, ov1)
        SI, SO = (si0, si1), (so0, so1)

        def start_in(k, b):
            row0 = base + k * _CS
            pltpu.async_copy(x_hbm.at[pl.ds(row0, _CS)], X[b], SI[b])
            pltpu.async_copy(pos_hbm.at[pl.ds(row0, _CS)], P[b], SI[b])

        def wait_in(b):
            pltpu.make_async_copy(x_hbm.at[pl.ds(0, _CS)], X[b], SI[b]).wait()
            pltpu.make_async_copy(pos_hbm.at[pl.ds(0, _CS)], P[b], SI[b]).wait()

        def start_out(k, b):
            row0 = base + k * _CS
            pltpu.async_copy(O[b], out_hbm.at[pl.ds(row0, _CS)], SO[b])

        def wait_out(b):
            pltpu.make_async_copy(O[b], out_hbm.at[pl.ds(0, _CS)], SO[b]).wait()

        def compute(b):
            xv, pv, ov = X[b], P[b], O[b]

            @plsc.parallel_loop(0, _CS * d_vecs, unroll=8)
            def _(j):
                sl = j // d_vecs
                off = pl.multiple_of((j % d_vecs) * _L, _L)
                pvec = pv[sl, pl.ds(off, _L)]
                for bb in range(B):
                    ov[sl, bb, pl.ds(off, _L)] = (
                        xv[sl, bb, pl.ds(off, _L)] + pvec
                    )

        start_in(0, 0)
        start_in(1, 1)

        def pair(k2, carry):
            for b in range(2):
                k = 2 * k2 + b
                wait_in(b)

                @pl.when(k >= 2)
                def _():
                    wait_out(b)

                compute(b)
                start_out(k, b)

                @pl.when(k + 2 < n_chunks)
                def _():
                    start_in(k + 2, b)
            return carry

        lax.fori_loop(0, n_chunks // 2, pair, 0)
        wait_out(0)
        wait_out(1)

    return run(x, pos_table[:S])
